# Initial kernel scaffold; baseline (speedup 1.0000x reference)
import jax
import jax.numpy as jnp
from jax.experimental import pallas as pl


def kernel(x, edge_index, batch, W1a, b1a, g1, be1, W1b, b1b, W2a, b2a, g2, be2, W2b, b2b, Wc1, bc1, Wc2, bc2):
    raise NotImplementedError("write your pallas kernel here")


# speedup vs baseline: 8.8528x; 8.8528x over previous
import functools

import jax
import jax.numpy as jnp
from jax import lax
from jax.experimental import pallas as pl
from jax.experimental.pallas import tpu as pltpu
from jax.experimental.pallas import tpu_sc as plsc

_N = 10000
_F = 128
_H = 32
_B = 16
_E = 320000

_NC = 2
_NS = 16
_CH = 128
_K = -(-_E // (_NC * _NS * _CH))
_EPAD = _NC * _NS * _K * _CH
_NPAD = -(-(_N + 1) // (_NS * 8)) * _NS * 8
_RPW = _NPAD // _NS


def _segment_sum_sc(y, src3, dst3, zpad):
    mesh = plsc.VectorSubcoreMesh(core_axis_name="c", subcore_axis_name="s")

    @functools.partial(
        pl.kernel,
        out_type=jax.ShapeDtypeStruct((_NC, _NPAD, _H), jnp.float32),
        mesh=mesh,
        scratch_types=[
            pltpu.VMEM((_K, _CH), jnp.int32),
            pltpu.VMEM((_K, _CH), jnp.int32),
            pltpu.VMEM((_CH, _H), jnp.float32),
            pltpu.VMEM_SHARED((_NPAD, _H), jnp.float32),
            pltpu.SemaphoreType.DMA,
        ],
        compiler_params=pltpu.CompilerParams(use_tc_tiling_on_sc=False),
    )
    def seg_kernel(y_hbm, src_hbm, dst_hbm, z_hbm, out_hbm,
                   src_v, dst_v, rows_v, acc_sh, sem):
        cid = lax.axis_index("c")
        sid = lax.axis_index("s")
        r0 = sid * _RPW
        pltpu.sync_copy(z_hbm.at[pl.ds(r0, _RPW)], acc_sh.at[pl.ds(r0, _RPW)])
        pltpu.sync_copy(src_hbm.at[cid, sid], src_v)
        pltpu.sync_copy(dst_hbm.at[cid, sid], dst_v)
        plsc.subcore_barrier()

        def body(j, carry):
            pltpu.async_copy(y_hbm.at[src_v.at[j]], rows_v, sem).wait()
            pltpu.sync_copy(rows_v, acc_sh.at[dst_v.at[j]], add=True)
            return carry

        lax.fori_loop(0, _K, body, 0)
        plsc.subcore_barrier()
        pltpu.sync_copy(acc_sh.at[pl.ds(r0, _RPW)],
                        out_hbm.at[cid, pl.ds(r0, _RPW)])

    return seg_kernel(y, src3, dst3, zpad)


def _proj_tc(x, W):
    def body(x_ref, w_ref, y_ref):
        y_ref[...] = jnp.dot(x_ref[...], w_ref[...],
                             preferred_element_type=jnp.float32)

    return pl.pallas_call(
        body, out_shape=jax.ShapeDtypeStruct((_N, _H), jnp.float32))(x, W)


def _bn_mlp(h, g, be, Wb, bb):
    mu = jnp.mean(h, axis=0)
    var = jnp.mean((h - mu) ** 2, axis=0)
    h = (h - mu) * lax.rsqrt(var + 1e-5) * g + be
    h = jnp.maximum(h, 0.0)
    return jnp.dot(h, Wb, preferred_element_type=jnp.float32) + bb


def _mid_tc(agg, y1, b1a, g1, be1, W1b, b1b, W2a):
    def body(agg_ref, y1_ref, b1a_ref, g1_ref, be1_ref, w1b_ref, b1b_ref,
             w2a_ref, y2_ref):
        h = y1_ref[...] + agg_ref[0, :_N, :] + agg_ref[1, :_N, :] + b1a_ref[...]
        h = _bn_mlp(h, g1_ref[...], be1_ref[...], w1b_ref[...], b1b_ref[...])
        h = jnp.maximum(h, 0.0)
        y2_ref[...] = jnp.dot(h, w2a_ref[...], preferred_element_type=jnp.float32)

    return pl.pallas_call(
        body, out_shape=jax.ShapeDtypeStruct((_N, _H), jnp.float32))(
            agg, y1, b1a, g1, be1, W1b, b1b, W2a)


def _head_tc(agg, y2, batch2d, b2a, g2, be2, W2b, b2b, Wc1, bc1, Wc2, bc2):
    def body(agg_ref, y2_ref, bat_ref, b2a_ref, g2_ref, be2_ref, w2b_ref,
             b2b_ref, wc1_ref, bc1_ref, wc2_ref, bc2_ref, out_ref):
        h = y2_ref[...] + agg_ref[0, :_N, :] + agg_ref[1, :_N, :] + b2a_ref[...]
        h = _bn_mlp(h, g2_ref[...], be2_ref[...], w2b_ref[...], b2b_ref[...])
        bat = bat_ref[...]
        means = []
        maxs = []
        neg = jnp.float32(-jnp.inf)
        for b in range(_B):
            m = bat == b
            cnt = jnp.sum(m.astype(jnp.float32))
            s = jnp.sum(jnp.where(m, h, 0.0), axis=0)
            means.append(s / jnp.maximum(cnt, 1.0))
            maxs.append(jnp.max(jnp.where(m, h, neg), axis=0))
        x_mean = jnp.stack(means, axis=0)
        x_max = jnp.stack(maxs, axis=0)
        z = (jnp.dot(x_mean, wc1_ref[:_H, :], preferred_element_type=jnp.float32)
             + jnp.dot(x_max, wc1_ref[_H:, :], preferred_element_type=jnp.float32)
             + bc1_ref[...])
        z = jnp.maximum(z, 0.0)
        out_ref[...] = (jnp.dot(z, wc2_ref[...], preferred_element_type=jnp.float32)
                        + bc2_ref[...])

    return pl.pallas_call(
        body, out_shape=jax.ShapeDtypeStruct((_B, 2), jnp.float32))(
            agg, y2, batch2d, b2a, g2, be2, W2b, b2b, Wc1, bc1, Wc2, bc2)


def kernel(x, edge_index, batch, W1a, b1a, g1, be1, W1b, b1b,
           W2a, b2a, g2, be2, W2b, b2b, Wc1, bc1, Wc2, bc2):
    src, dst = edge_index[0], edge_index[1]
    pad = _EPAD - _E
    src3 = jnp.concatenate([src, jnp.zeros((pad,), jnp.int32)]
                           ).reshape(_NC, _NS, _K, _CH)
    dst3 = jnp.concatenate([dst, jnp.full((pad,), _N, jnp.int32)]
                           ).reshape(_NC, _NS, _K, _CH)
    zpad = jnp.zeros((_NPAD, _H), jnp.float32)
    batch2d = batch.reshape(_N, 1)

    y1 = _proj_tc(x, W1a)
    agg1 = _segment_sum_sc(y1, src3, dst3, zpad)
    y2 = _mid_tc(agg1, y1, b1a, g1, be1, W1b, b1b, W2a)
    agg2 = _segment_sum_sc(y2, src3, dst3, zpad)
    return _head_tc(agg2, y2, batch2d, b2a, g2, be2, W2b, b2b,
                    Wc1, bc1, Wc2, bc2)
